# baseline (speedup 1.0000x reference)
import functools

import jax
import jax.numpy as jnp
from jax import lax
from jax.experimental import pallas as pl
from jax.experimental.pallas import tpu as pltpu
from jax.experimental.pallas import tpu_sc as plsc

N_NODES = 10000
D = 128
E = 320000
NC = 2
NS = 16
NW = NC * NS
EW = E // NW
CHUNK = 64
N_CHUNKS = EW // CHUNK
TAIL = EW - N_CHUNKS * CHUNK
GROUPS = D // 16
ZROWS = 624


def _mesh():
    return plsc.VectorSubcoreMesh(core_axis_name="c", subcore_axis_name="s")


NBUF = 5


@functools.partial(
    pl.kernel,
    out_type=jax.ShapeDtypeStruct((NC, N_NODES, D), jnp.float32),
    mesh=_mesh(),
    scratch_types=[
        pltpu.VMEM_SHARED((N_NODES, D), jnp.float32),
        pltpu.VMEM((NBUF, CHUNK), jnp.int32),
        pltpu.VMEM((NBUF, CHUNK), jnp.int32),
        pltpu.VMEM((NBUF, CHUNK), jnp.float32),
        pltpu.VMEM((NBUF, CHUNK, D), jnp.float32),
        pltpu.VMEM((TAIL,), jnp.int32),
        pltpu.VMEM((TAIL,), jnp.int32),
        pltpu.VMEM((TAIL,), jnp.float32),
        pltpu.VMEM((TAIL, D), jnp.float32),
        pltpu.SemaphoreType.DMA((NBUF,)),
        pltpu.SemaphoreType.DMA((NBUF,)),
        pltpu.SemaphoreType.DMA((NBUF,)),
    ],
)
def _propagate(table_hbm, src_hbm, dst_hbm, w_hbm, out_hbm,
               acc, src_v, dst_v, w_v, rows_v, tsrc, tdst, tw, trows,
               isem, gsem, ssem):
    c = lax.axis_index("c")
    s = lax.axis_index("s")
    wid = c * NS + s
    ebase = wid * EW

    zvec = jnp.zeros((16,), jnp.float32)

    def zero_row(r, carry):
        for g in range(GROUPS):
            rows_v[0, r, pl.ds(g * 16, 16)] = zvec
        return carry

    lax.fori_loop(0, CHUNK, zero_row, 0)

    zbase = s * ZROWS

    def zero_acc(i, carry):
        pltpu.sync_copy(rows_v.at[0], acc.at[pl.ds(zbase + i * CHUNK, CHUNK)])
        return carry

    lax.fori_loop(0, ZROWS // CHUNK, zero_acc, 0)

    @pl.when(s < NS - 1)
    def _():
        pltpu.sync_copy(rows_v.at[0, pl.ds(0, 48)],
                        acc.at[pl.ds(zbase + 9 * CHUNK, 48)])

    @pl.when(s == NS - 1)
    def _():
        pltpu.sync_copy(rows_v.at[0], acc.at[pl.ds(zbase + 9 * CHUNK, CHUNK)])

    def issue_idx(chunk, b):
        off = ebase + chunk * CHUNK
        pltpu.async_copy(src_hbm.at[pl.ds(off, CHUNK)], src_v.at[b],
                         isem.at[b])
        pltpu.async_copy(dst_hbm.at[pl.ds(off, CHUNK)], dst_v.at[b],
                         isem.at[b])
        pltpu.async_copy(w_hbm.at[pl.ds(off, CHUNK)], w_v.at[b], isem.at[b])

    def wait_idx(chunk, b):
        off = ebase + chunk * CHUNK
        pltpu.make_async_copy(src_hbm.at[pl.ds(off, CHUNK)], src_v.at[b],
                              isem.at[b]).wait()
        pltpu.make_async_copy(dst_hbm.at[pl.ds(off, CHUNK)], dst_v.at[b],
                              isem.at[b]).wait()
        pltpu.make_async_copy(w_hbm.at[pl.ds(off, CHUNK)], w_v.at[b],
                              isem.at[b]).wait()

    def issue_gather(b):
        pltpu.async_copy(table_hbm.at[src_v.at[b]], rows_v.at[b], gsem.at[b])

    def wait_gather(b):
        pltpu.make_async_copy(table_hbm.at[src_v.at[b]], rows_v.at[b],
                              gsem.at[b]).wait()

    def wait_scatter(b):
        pltpu.make_async_copy(rows_v.at[b], acc.at[dst_v.at[b]],
                              ssem.at[b]).wait()

    def process(i, b):
        wait_gather(b)

        @plsc.parallel_loop(0, CHUNK // 16, unroll=2)
        def blk_body(bb):
            w16 = w_v[b, pl.ds(bb * 16, 16)]
            for r in range(16):
                wspl = w16[jnp.full((16,), r, jnp.int32)]
                for gg in range(GROUPS):
                    sl = pl.ds(gg * 16, 16)
                    rows_v[b, bb * 16 + r, sl] = (
                        rows_v[b, bb * 16 + r, sl] * wspl)
        pltpu.async_copy(rows_v.at[b], acc.at[dst_v.at[b]], ssem.at[b],
                         add=True)

    for b in range(3):
        issue_idx(b, b)
    for b in range(2):
        wait_idx(b, b)
        issue_gather(b)

    plsc.subcore_barrier()

    def group_body(g, carry):
        for b in range(NBUF):
            i = g * NBUF + b

            @pl.when(i + 3 < N_CHUNKS)
            def _():
                @pl.when(i + 3 >= NBUF)
                def _():
                    wait_scatter((b + 3) % NBUF)

                issue_idx(i + 3, (b + 3) % NBUF)

            @pl.when(i + 2 < N_CHUNKS)
            def _():
                wait_idx(i + 2, (b + 2) % NBUF)
                issue_gather((b + 2) % NBUF)

            process(i, b)
        return carry

    lax.fori_loop(0, (N_CHUNKS - 1) // NBUF, group_body, 0)
    process(N_CHUNKS - 1, 0)
    for b in range(NBUF):
        wait_scatter(b)

    toff = ebase + N_CHUNKS * CHUNK
    pltpu.sync_copy(src_hbm.at[pl.ds(toff, TAIL)], tsrc)
    pltpu.sync_copy(dst_hbm.at[pl.ds(toff, TAIL)], tdst)
    pltpu.sync_copy(w_hbm.at[pl.ds(toff, TAIL)], tw)
    pltpu.async_copy(table_hbm.at[tsrc], trows, gsem.at[0]).wait()
    tw16 = tw[pl.ds(0, 16)]
    for r in range(TAIL):
        wspl = jnp.full((16,), tw16[r], jnp.float32)
        for gg in range(GROUPS):
            sl = pl.ds(gg * 16, 16)
            trows[r, sl] = trows[r, sl] * wspl
    pltpu.sync_copy(trows, acc.at[tdst], add=True)
    plsc.subcore_barrier()

    @pl.when(s < NS - 1)
    def _():
        pltpu.sync_copy(acc.at[pl.ds(zbase, ZROWS)],
                        out_hbm.at[c, pl.ds(zbase, ZROWS)])

    @pl.when(s == NS - 1)
    def _():
        pltpu.sync_copy(acc.at[pl.ds(zbase, ZROWS + 16)],
                        out_hbm.at[c, pl.ds(zbase, ZROWS + 16)])


CR = 80
NCHUNK_COMB = N_NODES // CR


def _combine_body(srcs, out_hbm, acc_v, tmp_v, scale):
    c = lax.axis_index("c")
    s = lax.axis_index("s")
    wid = c * NS + s
    n_iter = (NCHUNK_COMB - 1) // NW + 1

    def do_chunk(j, carry):
        cid = j * NW + wid

        @pl.when(cid < NCHUNK_COMB)
        def _():
            rows = pl.ds(cid * CR, CR)
            pltpu.sync_copy(srcs[0].at[rows], acc_v)
            for k, src in enumerate(srcs[1:]):
                last = k == len(srcs) - 2
                pltpu.sync_copy(src.at[rows], tmp_v)

                def add_row(r, inner, last=last):
                    for g in range(GROUPS):
                        sl = pl.ds(g * 16, 16)
                        v = acc_v[r, sl] + tmp_v[r, sl]
                        if last and scale != 1.0:
                            v = v * scale
                        acc_v[r, sl] = v
                    return inner

                lax.fori_loop(0, CR, add_row, 0)
            pltpu.sync_copy(acc_v, out_hbm.at[rows])

        return carry

    lax.fori_loop(0, n_iter, do_chunk, 0)


@functools.partial(
    pl.kernel,
    out_type=jax.ShapeDtypeStruct((N_NODES, D), jnp.float32),
    mesh=_mesh(),
    scratch_types=[
        pltpu.VMEM((CR, D), jnp.float32),
        pltpu.VMEM((CR, D), jnp.float32),
    ],
)
def _combine2(p_hbm, out_hbm, acc_v, tmp_v):
    _combine_body([p_hbm.at[0], p_hbm.at[1]], out_hbm, acc_v, tmp_v, 1.0)


@functools.partial(
    pl.kernel,
    out_type=jax.ShapeDtypeStruct((N_NODES, D), jnp.float32),
    mesh=_mesh(),
    scratch_types=[
        pltpu.VMEM((CR, D), jnp.float32),
        pltpu.VMEM((CR, D), jnp.float32),
    ],
)
def _final4(p_hbm, t0_hbm, t1_hbm, t2_hbm, out_hbm, acc_v, tmp_v):
    _combine_body([p_hbm.at[0], p_hbm.at[1], t0_hbm, t1_hbm, t2_hbm],
                  out_hbm, acc_v, tmp_v, 0.25)


@jax.jit
def kernel(user_emb, item_emb, edge_index, edge_weight):
    n_users = user_emb.shape[0]
    t0 = jnp.concatenate([user_emb, item_emb], axis=0)
    src = edge_index[0]
    dst = edge_index[1]

    p1 = _propagate(t0, src, dst, edge_weight)
    t1 = _combine2(p1)
    p2 = _propagate(t1, src, dst, edge_weight)
    t2 = _combine2(p2)
    p3 = _propagate(t2, src, dst, edge_weight)
    mean = _final4(p3, t0, t1, t2)
    return mean[:n_users], mean[n_users:]


# speedup vs baseline: 1.0451x; 1.0451x over previous
import functools

import jax
import jax.numpy as jnp
from jax import lax
from jax.experimental import pallas as pl
from jax.experimental.pallas import tpu as pltpu
from jax.experimental.pallas import tpu_sc as plsc

N_NODES = 10000
D = 128
E = 320000
NC = 2
NS = 16
NW = NC * NS
EW = E // NW
CHUNK = 64
N_CHUNKS = EW // CHUNK
TAIL = EW - N_CHUNKS * CHUNK
GROUPS = D // 16
ZROWS = 624


def _mesh():
    return plsc.VectorSubcoreMesh(core_axis_name="c", subcore_axis_name="s")


NBUF = 5


@functools.partial(
    pl.kernel,
    out_type=jax.ShapeDtypeStruct((NC, N_NODES, D), jnp.float32),
    mesh=_mesh(),
    scratch_types=[
        pltpu.VMEM_SHARED((N_NODES, D), jnp.float32),
        pltpu.VMEM((NBUF, CHUNK), jnp.int32),
        pltpu.VMEM((NBUF, CHUNK), jnp.int32),
        pltpu.VMEM((NBUF, CHUNK), jnp.float32),
        pltpu.VMEM((NBUF, CHUNK, D), jnp.float32),
        pltpu.VMEM((TAIL,), jnp.int32),
        pltpu.VMEM((TAIL,), jnp.int32),
        pltpu.VMEM((TAIL,), jnp.float32),
        pltpu.VMEM((TAIL, D), jnp.float32),
        pltpu.SemaphoreType.DMA((NBUF,)),
        pltpu.SemaphoreType.DMA((NBUF,)),
        pltpu.SemaphoreType.DMA((NBUF,)),
    ],
)
def _propagate(table_hbm, src_hbm, dst_hbm, w_hbm, out_hbm,
               acc, src_v, dst_v, w_v, rows_v, tsrc, tdst, tw, trows,
               isem, gsem, ssem):
    c = lax.axis_index("c")
    s = lax.axis_index("s")
    wid = c * NS + s
    ebase = wid * EW

    zvec = jnp.zeros((16,), jnp.float32)

    def zero_row(r, carry):
        for g in range(GROUPS):
            rows_v[0, r, pl.ds(g * 16, 16)] = zvec
        return carry

    lax.fori_loop(0, CHUNK, zero_row, 0)

    zbase = s * ZROWS

    def zero_acc(i, carry):
        pltpu.sync_copy(rows_v.at[0], acc.at[pl.ds(zbase + i * CHUNK, CHUNK)])
        return carry

    lax.fori_loop(0, ZROWS // CHUNK, zero_acc, 0)

    @pl.when(s < NS - 1)
    def _():
        pltpu.sync_copy(rows_v.at[0, pl.ds(0, 48)],
                        acc.at[pl.ds(zbase + 9 * CHUNK, 48)])

    @pl.when(s == NS - 1)
    def _():
        pltpu.sync_copy(rows_v.at[0], acc.at[pl.ds(zbase + 9 * CHUNK, CHUNK)])

    def issue_idx(chunk, b):
        off = ebase + chunk * CHUNK
        pltpu.async_copy(src_hbm.at[pl.ds(off, CHUNK)], src_v.at[b],
                         isem.at[b])
        pltpu.async_copy(dst_hbm.at[pl.ds(off, CHUNK)], dst_v.at[b],
                         isem.at[b])
        pltpu.async_copy(w_hbm.at[pl.ds(off, CHUNK)], w_v.at[b], isem.at[b])

    def wait_idx(chunk, b):
        off = ebase + chunk * CHUNK
        pltpu.make_async_copy(src_hbm.at[pl.ds(off, CHUNK)], src_v.at[b],
                              isem.at[b]).wait()
        pltpu.make_async_copy(dst_hbm.at[pl.ds(off, CHUNK)], dst_v.at[b],
                              isem.at[b]).wait()
        pltpu.make_async_copy(w_hbm.at[pl.ds(off, CHUNK)], w_v.at[b],
                              isem.at[b]).wait()

    def issue_gather(b):
        pltpu.async_copy(table_hbm.at[src_v.at[b]], rows_v.at[b], gsem.at[b])

    def wait_gather(b):
        pltpu.make_async_copy(table_hbm.at[src_v.at[b]], rows_v.at[b],
                              gsem.at[b]).wait()

    def wait_scatter(b):
        pltpu.make_async_copy(rows_v.at[b], acc.at[dst_v.at[b]],
                              ssem.at[b]).wait()

    def process(i, b):
        wait_gather(b)

        def blk_body(bb, inner):
            w16 = w_v[b, pl.ds(bb * 16, 16)]
            for r in range(16):
                wspl = w16[jnp.full((16,), r, jnp.int32)]
                for gg in range(GROUPS):
                    sl = pl.ds(gg * 16, 16)
                    rows_v[b, bb * 16 + r, sl] = (
                        rows_v[b, bb * 16 + r, sl] * wspl)
            return inner

        lax.fori_loop(0, CHUNK // 16, blk_body, 0)
        pltpu.async_copy(rows_v.at[b], acc.at[dst_v.at[b]], ssem.at[b],
                         add=True)

    for b in range(3):
        issue_idx(b, b)
    for b in range(2):
        wait_idx(b, b)
        issue_gather(b)

    plsc.subcore_barrier()

    def group_body(g, carry):
        for b in range(NBUF):
            i = g * NBUF + b

            @pl.when(i + 3 < N_CHUNKS)
            def _():
                @pl.when(i + 3 >= NBUF)
                def _():
                    wait_scatter((b + 3) % NBUF)

                issue_idx(i + 3, (b + 3) % NBUF)

            @pl.when(i + 2 < N_CHUNKS)
            def _():
                wait_idx(i + 2, (b + 2) % NBUF)
                issue_gather((b + 2) % NBUF)

            process(i, b)
        return carry

    lax.fori_loop(0, (N_CHUNKS - 1) // NBUF, group_body, 0)
    process(N_CHUNKS - 1, 0)
    for b in range(NBUF):
        wait_scatter(b)

    toff = ebase + N_CHUNKS * CHUNK
    pltpu.sync_copy(src_hbm.at[pl.ds(toff, TAIL)], tsrc)
    pltpu.sync_copy(dst_hbm.at[pl.ds(toff, TAIL)], tdst)
    pltpu.sync_copy(w_hbm.at[pl.ds(toff, TAIL)], tw)
    pltpu.async_copy(table_hbm.at[tsrc], trows, gsem.at[0]).wait()
    tw16 = tw[pl.ds(0, 16)]
    for r in range(TAIL):
        wspl = jnp.full((16,), tw16[r], jnp.float32)
        for gg in range(GROUPS):
            sl = pl.ds(gg * 16, 16)
            trows[r, sl] = trows[r, sl] * wspl
    pltpu.sync_copy(trows, acc.at[tdst], add=True)
    plsc.subcore_barrier()

    @pl.when(s < NS - 1)
    def _():
        pltpu.sync_copy(acc.at[pl.ds(zbase, ZROWS)],
                        out_hbm.at[c, pl.ds(zbase, ZROWS)])

    @pl.when(s == NS - 1)
    def _():
        pltpu.sync_copy(acc.at[pl.ds(zbase, ZROWS + 16)],
                        out_hbm.at[c, pl.ds(zbase, ZROWS + 16)])


CR = 80
NCHUNK_COMB = N_NODES // CR


def _combine_body(srcs, out_hbm, acc_v, tmp_v, scale):
    c = lax.axis_index("c")
    s = lax.axis_index("s")
    wid = c * NS + s
    n_iter = (NCHUNK_COMB - 1) // NW + 1

    def do_chunk(j, carry):
        cid = j * NW + wid

        @pl.when(cid < NCHUNK_COMB)
        def _():
            rows = pl.ds(cid * CR, CR)
            pltpu.sync_copy(srcs[0].at[rows], acc_v)
            for k, src in enumerate(srcs[1:]):
                last = k == len(srcs) - 2
                pltpu.sync_copy(src.at[rows], tmp_v)

                def add_row(r, inner, last=last):
                    for g in range(GROUPS):
                        sl = pl.ds(g * 16, 16)
                        v = acc_v[r, sl] + tmp_v[r, sl]
                        if last and scale != 1.0:
                            v = v * scale
                        acc_v[r, sl] = v
                    return inner

                lax.fori_loop(0, CR, add_row, 0)
            pltpu.sync_copy(acc_v, out_hbm.at[rows])

        return carry

    lax.fori_loop(0, n_iter, do_chunk, 0)


@functools.partial(
    pl.kernel,
    out_type=jax.ShapeDtypeStruct((N_NODES, D), jnp.float32),
    mesh=_mesh(),
    scratch_types=[
        pltpu.VMEM((CR, D), jnp.float32),
        pltpu.VMEM((CR, D), jnp.float32),
    ],
)
def _combine2(p_hbm, out_hbm, acc_v, tmp_v):
    _combine_body([p_hbm.at[0], p_hbm.at[1]], out_hbm, acc_v, tmp_v, 1.0)


@functools.partial(
    pl.kernel,
    out_type=jax.ShapeDtypeStruct((N_NODES, D), jnp.float32),
    mesh=_mesh(),
    scratch_types=[
        pltpu.VMEM((CR, D), jnp.float32),
        pltpu.VMEM((CR, D), jnp.float32),
    ],
)
def _final4(p_hbm, t0_hbm, t1_hbm, t2_hbm, out_hbm, acc_v, tmp_v):
    _combine_body([p_hbm.at[0], p_hbm.at[1], t0_hbm, t1_hbm, t2_hbm],
                  out_hbm, acc_v, tmp_v, 0.25)


@jax.jit
def kernel(user_emb, item_emb, edge_index, edge_weight):
    n_users = user_emb.shape[0]
    t0 = jnp.concatenate([user_emb, item_emb], axis=0)
    src = edge_index[0]
    dst = edge_index[1]

    p1 = _propagate(t0, src, dst, edge_weight)
    t1 = _combine2(p1)
    p2 = _propagate(t1, src, dst, edge_weight)
    t2 = _combine2(p2)
    p3 = _propagate(t2, src, dst, edge_weight)
    mean = _final4(p3, t0, t1, t2)
    return mean[:n_users], mean[n_users:]
